# baseline (speedup 1.0000x reference)
from functools import partial

import jax
import jax.numpy as jnp
from jax.experimental import pallas as pl
from jax.experimental.pallas import tpu as pltpu

ENERGY_MOMENTUM = 0.9
BASE_PROB = 0.1
ENERGY_SCALE = 2.0
LAYER_IDX = 0

_CHUNK = 256


def _var_probs_kernel(x_ref, he_ref, probs_ref, acc_ref, *, n_steps, n_total):
    i = pl.program_id(0)

    @pl.when(i == 0)
    def _init():
        acc_ref[...] = jnp.zeros_like(acc_ref)

    x = x_ref[...]
    acc_ref[0, :, :] += jnp.sum(x, axis=0)
    acc_ref[1, :, :] += jnp.sum(x * x, axis=0)

    @pl.when(i == n_steps - 1)
    def _epilogue():
        ssum = jnp.sum(acc_ref[0, :, :], axis=1)
        ssq = jnp.sum(acc_ref[1, :, :], axis=1)
        n = jnp.float32(n_total)
        head_var = (ssq - ssum * ssum / n) / (n - 1.0)
        mx = jnp.max(head_var)
        head_var = jnp.where(mx > 0, head_var / (mx + 1e-08), head_var)

        he = he_ref[...]
        new_row = ENERGY_MOMENTUM * he[LAYER_IDX, :] + (1.0 - ENERGY_MOMENTUM) * head_var
        row_ids = jax.lax.broadcasted_iota(jnp.int32, he.shape, 0)
        new_energy = jnp.where(row_ids == LAYER_IDX, new_row[None, :], he)

        inv = 1.0 / (new_energy + 0.1)
        inv = inv / jnp.max(inv)
        probs = BASE_PROB * (1.0 + ENERGY_SCALE * inv)
        probs_ref[...] = jnp.clip(probs, 0.0, 1.0)


def kernel(output, head_energy):
    B, T, H, D = output.shape
    x = output.reshape(B * T, H, D)
    rows = B * T
    n_steps = rows // _CHUNK
    n_total = rows * D

    return pl.pallas_call(
        partial(_var_probs_kernel, n_steps=n_steps, n_total=n_total),
        grid=(n_steps,),
        in_specs=[
            pl.BlockSpec((_CHUNK, H, D), lambda i: (i, 0, 0)),
            pl.BlockSpec(head_energy.shape, lambda i: (0, 0)),
        ],
        out_specs=pl.BlockSpec(head_energy.shape, lambda i: (0, 0)),
        out_shape=jax.ShapeDtypeStruct(head_energy.shape, jnp.float32),
        scratch_shapes=[pltpu.VMEM((2, H, D), jnp.float32)],
    )(x, head_energy)


# speedup vs baseline: 1.1706x; 1.1706x over previous
from functools import partial

import jax
import jax.numpy as jnp
from jax.experimental import pallas as pl
from jax.experimental.pallas import tpu as pltpu

ENERGY_MOMENTUM = 0.9
BASE_PROB = 0.1
ENERGY_SCALE = 2.0
LAYER_IDX = 0

_CHUNK = 1024


def _var_probs_kernel(x_ref, he_ref, probs_ref, acc_ref, *, n_steps, n_total):
    i = pl.program_id(0)

    @pl.when(i == 0)
    def _init():
        acc_ref[...] = jnp.zeros_like(acc_ref)

    x = x_ref[...]
    acc_ref[0, :, :] += jnp.sum(x, axis=0)
    acc_ref[1, :, :] += jnp.sum(x * x, axis=0)

    @pl.when(i == n_steps - 1)
    def _epilogue():
        ssum = jnp.sum(acc_ref[0, :, :], axis=1)
        ssq = jnp.sum(acc_ref[1, :, :], axis=1)
        n = jnp.float32(n_total)
        head_var = (ssq - ssum * ssum / n) / (n - 1.0)
        mx = jnp.max(head_var)
        head_var = jnp.where(mx > 0, head_var / (mx + 1e-08), head_var)

        he = he_ref[...]
        new_row = ENERGY_MOMENTUM * he[LAYER_IDX, :] + (1.0 - ENERGY_MOMENTUM) * head_var
        row_ids = jax.lax.broadcasted_iota(jnp.int32, he.shape, 0)
        new_energy = jnp.where(row_ids == LAYER_IDX, new_row[None, :], he)

        inv = 1.0 / (new_energy + 0.1)
        inv = inv / jnp.max(inv)
        probs = BASE_PROB * (1.0 + ENERGY_SCALE * inv)
        probs_ref[...] = jnp.clip(probs, 0.0, 1.0)


def kernel(output, head_energy):
    B, T, H, D = output.shape
    x = output.reshape(B * T, H, D)
    rows = B * T
    n_steps = rows // _CHUNK
    n_total = rows * D

    return pl.pallas_call(
        partial(_var_probs_kernel, n_steps=n_steps, n_total=n_total),
        grid=(n_steps,),
        in_specs=[
            pl.BlockSpec((_CHUNK, H, D), lambda i: (i, 0, 0)),
            pl.BlockSpec(head_energy.shape, lambda i: (0, 0)),
        ],
        out_specs=pl.BlockSpec(head_energy.shape, lambda i: (0, 0)),
        out_shape=jax.ShapeDtypeStruct(head_energy.shape, jnp.float32),
        scratch_shapes=[pltpu.VMEM((2, H, D), jnp.float32)],
    )(x, head_energy)


# speedup vs baseline: 1.2360x; 1.0558x over previous
from functools import partial

import jax
import jax.numpy as jnp
from jax.experimental import pallas as pl
from jax.experimental.pallas import tpu as pltpu

ENERGY_MOMENTUM = 0.9
BASE_PROB = 0.1
ENERGY_SCALE = 2.0
LAYER_IDX = 0

_CHUNK = 512


def _var_probs_kernel(x_ref, he_ref, probs_ref, acc_ref, *, n_steps, n_total):
    i = pl.program_id(0)

    @pl.when(i == 0)
    def _init():
        acc_ref[...] = jnp.zeros_like(acc_ref)

    x = x_ref[...]
    acc_ref[0, :, :] += jnp.sum(x, axis=0)
    acc_ref[1, :, :] += jnp.sum(x * x, axis=0)

    @pl.when(i == n_steps - 1)
    def _epilogue():
        ssum = jnp.sum(acc_ref[0, :, :], axis=1)
        ssq = jnp.sum(acc_ref[1, :, :], axis=1)
        n = jnp.float32(n_total)
        head_var = (ssq - ssum * ssum / n) / (n - 1.0)
        mx = jnp.max(head_var)
        head_var = jnp.where(mx > 0, head_var / (mx + 1e-08), head_var)

        he = he_ref[...]
        new_row = ENERGY_MOMENTUM * he[LAYER_IDX, :] + (1.0 - ENERGY_MOMENTUM) * head_var
        row_ids = jax.lax.broadcasted_iota(jnp.int32, he.shape, 0)
        new_energy = jnp.where(row_ids == LAYER_IDX, new_row[None, :], he)

        inv = 1.0 / (new_energy + 0.1)
        inv = inv / jnp.max(inv)
        probs = BASE_PROB * (1.0 + ENERGY_SCALE * inv)
        probs_ref[...] = jnp.clip(probs, 0.0, 1.0)


def kernel(output, head_energy):
    B, T, H, D = output.shape
    x = output.reshape(B * T, H, D)
    rows = B * T
    n_steps = rows // _CHUNK
    n_total = rows * D

    return pl.pallas_call(
        partial(_var_probs_kernel, n_steps=n_steps, n_total=n_total),
        grid=(n_steps,),
        in_specs=[
            pl.BlockSpec((_CHUNK, H, D), lambda i: (i, 0, 0)),
            pl.BlockSpec(head_energy.shape, lambda i: (0, 0)),
        ],
        out_specs=pl.BlockSpec(head_energy.shape, lambda i: (0, 0)),
        out_shape=jax.ShapeDtypeStruct(head_energy.shape, jnp.float32),
        scratch_shapes=[pltpu.VMEM((2, H, D), jnp.float32)],
    )(x, head_energy)
